# Initial kernel scaffold; baseline (speedup 1.0000x reference)
import functools

import jax
import jax.numpy as jnp
from jax import lax
from jax.experimental import pallas as pl
from jax.experimental.pallas import tpu as pltpu
from jax.experimental.pallas import tpu_sc as plsc

N = 10000
E = 320000
D = 128

NC = 2
NS = 16
L = 16

CHUNK = 128
E_PAD = 323584
EPS_DEG = E_PAD // (NC * NS)
NCH_DEG = EPS_DEG // CHUNK
EPS_MAIN = E_PAD // NS
NCH_MAIN = EPS_MAIN // CHUNK
RPS = N // NS
NH = 10240
NV = N + L
TOFF = N + 8

_mesh = plsc.VectorSubcoreMesh(core_axis_name="c", subcore_axis_name="s")


@functools.partial(
    pl.kernel,
    out_type=jax.ShapeDtypeStruct((NC * NS, NH), jnp.float32),
    mesh=_mesh,
    scratch_types=[
        pltpu.VMEM((CHUNK,), jnp.int32),
        pltpu.VMEM((NH,), jnp.float32),
    ],
)
def _deg_kernel(colp_hbm, degp_hbm, colv, hist):
    c = lax.axis_index("c")
    s = lax.axis_index("s")
    wid = c * NS + s

    def zbody(i, carry):
        hist[pl.ds(i * L, L)] = jnp.zeros((L,), jnp.float32)
        return carry

    lax.fori_loop(0, NH // L, zbody, 0)

    ones16 = jnp.ones((L,), jnp.float32)

    def cbody(ch, carry):
        base = wid * EPS_DEG + ch * CHUNK
        pltpu.sync_copy(colp_hbm.at[pl.ds(base, CHUNK)], colv)
        for i in range(CHUNK // L):
            c16 = colv[pl.ds(i * L, L)]
            plsc.addupdate_scatter(hist, [c16], ones16)
        return carry

    lax.fori_loop(0, NCH_DEG, cbody, 0)
    pltpu.sync_copy(hist, degp_hbm.at[wid])


RN = 2000


def _prep_body(degp_ref, x_ref, mask_ref, a_ref, b_ref, dis_ref):
    deg = jnp.sum(degp_ref[...], axis=0)
    dis = jnp.where(deg > 0, lax.rsqrt(deg), 0.0)
    disc = dis[:, None]
    xc = x_ref[...]
    xc = jnp.where(jnp.isnan(xc), 0.0, xc)
    m = mask_ref[...]
    a_ref[...] = disc * m * xc
    b_ref[...] = disc * m
    dis_ref[...] = jnp.broadcast_to(disc, (RN, D))


_prep = pl.pallas_call(
    _prep_body,
    grid=(N // RN,),
    in_specs=[
        pl.BlockSpec((NC * NS, RN), lambda i: (0, i)),
        pl.BlockSpec((RN, D), lambda i: (i, 0)),
        pl.BlockSpec((RN, D), lambda i: (i, 0)),
    ],
    out_specs=[
        pl.BlockSpec((RN, D), lambda i: (i, 0)),
        pl.BlockSpec((RN, D), lambda i: (i, 0)),
        pl.BlockSpec((RN, D), lambda i: (i, 0)),
    ],
    out_shape=[
        jax.ShapeDtypeStruct((N, D), jnp.float32),
        jax.ShapeDtypeStruct((N, D), jnp.float32),
        jax.ShapeDtypeStruct((N, D), jnp.float32),
    ],
)


@functools.partial(
    pl.kernel,
    out_type=(
        jax.ShapeDtypeStruct((2 * N, D), jnp.float32),
        jax.ShapeDtypeStruct((NC * NS, NV), jnp.float32),
    ),
    mesh=_mesh,
    scratch_types=[
        pltpu.VMEM_SHARED((N, D), jnp.float32),
        pltpu.VMEM((CHUNK,), jnp.int32),
        pltpu.VMEM((CHUNK,), jnp.int32),
        pltpu.VMEM((CHUNK, D), jnp.float32),
        pltpu.VMEM((NV,), jnp.float32),
        pltpu.VMEM((NV,), jnp.float32),
        pltpu.SemaphoreType.DMA,
    ],
)
def _main_kernel(col2_hbm, rowp_hbm, table_hbm, dis_hbm, zeros_hbm,
                 sums_hbm, s1p_hbm,
                 acc, colv, rowv, rowsv, disv, histv, sem):
    c = lax.axis_index("c")
    s = lax.axis_index("s")

    pltpu.sync_copy(zeros_hbm.at[pl.ds(s * RPS, RPS)],
                    acc.at[pl.ds(s * RPS, RPS)])
    pltpu.sync_copy(dis_hbm, disv)

    def zbody(i, carry):
        histv[pl.ds(i * L, L)] = jnp.zeros((L,), jnp.float32)
        return carry

    lax.fori_loop(0, NV // L, zbody, 0)
    plsc.subcore_barrier()

    coff = c * TOFF

    def cbody(ch, carry):
        base = s * EPS_MAIN + ch * CHUNK
        pltpu.sync_copy(col2_hbm.at[pl.ds(c * E_PAD + base, CHUNK)], colv)
        pltpu.sync_copy(rowp_hbm.at[pl.ds(base, CHUNK)], rowv)
        pltpu.async_copy(table_hbm.at[colv], rowsv, sem).wait()
        pltpu.sync_copy(rowsv, acc.at[rowv], add=True)

        @pl.when(lax.rem(ch, 2) == c)
        def _hist():
            for i in range(CHUNK // L):
                c16 = colv[pl.ds(i * L, L)] - coff
                r16 = rowv[pl.ds(i * L, L)]
                d16 = plsc.load_gather(disv, [c16])
                plsc.addupdate_scatter(histv, [r16], d16)

        return carry

    lax.fori_loop(0, NCH_MAIN, cbody, 0)
    plsc.subcore_barrier()
    pltpu.sync_copy(acc.at[pl.ds(s * RPS, RPS)],
                    sums_hbm.at[pl.ds(c * N + s * RPS, RPS)])
    pltpu.sync_copy(histv, s1p_hbm.at[c * NS + s])


def _final_body(s2_ref, s3_ref, s1p_ref, dis_ref, wt_ref, b_ref, o_ref):
    s1 = jnp.sum(s1p_ref[...], axis=0)
    num = dis_ref[...] * s1[:, None] * s2_ref[...]
    ratio = jnp.nan_to_num(num / s3_ref[...])
    o_ref[...] = (
        jnp.dot(ratio, wt_ref[...], preferred_element_type=jnp.float32)
        + b_ref[...]
    )


_final = pl.pallas_call(
    _final_body,
    grid=(N // RN,),
    in_specs=[
        pl.BlockSpec((RN, D), lambda i: (i, 0)),
        pl.BlockSpec((RN, D), lambda i: (i, 0)),
        pl.BlockSpec((NC * NS, RN), lambda i: (0, i)),
        pl.BlockSpec((RN, D), lambda i: (i, 0)),
        pl.BlockSpec((D, D), lambda i: (0, 0)),
        pl.BlockSpec((1, D), lambda i: (0, 0)),
    ],
    out_specs=pl.BlockSpec((RN, D), lambda i: (i, 0)),
    out_shape=jax.ShapeDtypeStruct((N, D), jnp.float32),
)


def kernel(x, edge_index, mask, W, b):
    row = edge_index[0]
    col = edge_index[1]
    pad = E_PAD - E
    colp = jnp.concatenate([col, jnp.full((pad,), N, jnp.int32)])
    rowp = jnp.concatenate([row, jnp.zeros((pad,), jnp.int32)])
    col2 = jnp.concatenate([colp, colp + TOFF])

    degp = _deg_kernel(colp)
    a, bm, discol = _prep(degp[:, :N], x, mask)

    z8 = jnp.zeros((8, D), jnp.float32)
    table = jnp.concatenate([a, z8, bm, z8], axis=0)
    dis1dp = jnp.pad(discol[:, 0], (0, NV - N))
    zeros_init = jnp.zeros((N, D), jnp.float32)

    sums, s1p = _main_kernel(col2, rowp, table, dis1dp, zeros_init)
    return _final(sums[:N], sums[N:], s1p[:, :N], discol, W.T, b[None])


# speedup vs baseline: 11.3480x; 11.3480x over previous
import functools

import jax
import jax.numpy as jnp
from jax import lax
from jax.experimental import pallas as pl
from jax.experimental.pallas import tpu as pltpu
from jax.experimental.pallas import tpu_sc as plsc

N = 10000
E = 320000
D = 128

NC = 2
NS = 16
L = 16

CHUNK = 128
E_PAD = 323584
EPS_DEG = E_PAD // (NC * NS)
NCH_DEG = EPS_DEG // CHUNK
EPS_MAIN = E_PAD // NS
NCH_MAIN = EPS_MAIN // CHUNK
NH = 10240
RPS = NH // NS
RN = 2048

_mesh = plsc.VectorSubcoreMesh(core_axis_name="c", subcore_axis_name="s")


@functools.partial(
    pl.kernel,
    out_type=jax.ShapeDtypeStruct((NC * NS, NH), jnp.float32),
    mesh=_mesh,
    scratch_types=[
        pltpu.VMEM((CHUNK,), jnp.int32),
        pltpu.VMEM((NH,), jnp.float32),
    ],
    compiler_params=pltpu.CompilerParams(needs_layout_passes=False),
)
def _deg_kernel(colp_hbm, degp_hbm, colv, hist):
    c = lax.axis_index("c")
    s = lax.axis_index("s")
    wid = c * NS + s

    def zbody(i, carry):
        hist[pl.ds(i * L, L)] = jnp.zeros((L,), jnp.float32)
        return carry

    lax.fori_loop(0, NH // L, zbody, 0)

    ones16 = jnp.ones((L,), jnp.float32)

    def cbody(ch, carry):
        base = wid * EPS_DEG + ch * CHUNK
        pltpu.sync_copy(colp_hbm.at[pl.ds(base, CHUNK)], colv)
        for i in range(CHUNK // L):
            c16 = colv[pl.ds(i * L, L)]
            plsc.addupdate_scatter(hist, [c16], ones16)
        return carry

    lax.fori_loop(0, NCH_DEG, cbody, 0)
    pltpu.sync_copy(hist, degp_hbm.at[wid])


def _prep_body(degp_ref, x_ref, mask_ref, a_ref, b_ref, dis_ref):
    deg = jnp.sum(degp_ref[...], axis=0)
    dis = jnp.where(deg > 0, lax.rsqrt(deg), 0.0)
    disc = dis[:, None]
    xc = x_ref[...]
    xc = jnp.where(jnp.isnan(xc), 0.0, xc)
    m = mask_ref[...]
    a_ref[...] = disc * m * xc
    b_ref[...] = disc * m
    dis_ref[...] = jnp.broadcast_to(disc, (RN, D))


_prep = pl.pallas_call(
    _prep_body,
    grid=(NH // RN,),
    in_specs=[
        pl.BlockSpec((NC * NS, RN), lambda i: (0, i)),
        pl.BlockSpec((RN, D), lambda i: (i, 0)),
        pl.BlockSpec((RN, D), lambda i: (i, 0)),
    ],
    out_specs=[
        pl.BlockSpec((RN, D), lambda i: (i, 0)),
        pl.BlockSpec((RN, D), lambda i: (i, 0)),
        pl.BlockSpec((RN, D), lambda i: (i, 0)),
    ],
    out_shape=[
        jax.ShapeDtypeStruct((NH, D), jnp.float32),
        jax.ShapeDtypeStruct((NH, D), jnp.float32),
        jax.ShapeDtypeStruct((NH, D), jnp.float32),
    ],
)


@functools.partial(
    pl.kernel,
    out_type=(
        jax.ShapeDtypeStruct((2 * NH, D), jnp.float32),
        jax.ShapeDtypeStruct((NC * NS, NH), jnp.float32),
    ),
    mesh=_mesh,
    scratch_types=[
        pltpu.VMEM_SHARED((NH, D), jnp.float32),
        pltpu.VMEM((CHUNK,), jnp.int32),
        pltpu.VMEM((CHUNK,), jnp.int32),
        pltpu.VMEM((CHUNK, D), jnp.float32),
        pltpu.VMEM((NH,), jnp.float32),
        pltpu.VMEM((NH,), jnp.float32),
        pltpu.SemaphoreType.DMA,
    ],
    compiler_params=pltpu.CompilerParams(needs_layout_passes=False),
)
def _main_kernel(col2_hbm, rowp_hbm, table_hbm, dis_hbm, zeros_hbm,
                 sums_hbm, s1p_hbm,
                 acc, colv, rowv, rowsv, disv, histv, sem):
    c = lax.axis_index("c")
    s = lax.axis_index("s")

    pltpu.sync_copy(zeros_hbm.at[pl.ds(s * RPS, RPS)],
                    acc.at[pl.ds(s * RPS, RPS)])
    pltpu.sync_copy(dis_hbm, disv)

    def zbody(i, carry):
        histv[pl.ds(i * L, L)] = jnp.zeros((L,), jnp.float32)
        return carry

    lax.fori_loop(0, NH // L, zbody, 0)
    plsc.subcore_barrier()

    coff = c * NH

    def cbody(ch, carry):
        base = s * EPS_MAIN + ch * CHUNK
        pltpu.sync_copy(col2_hbm.at[pl.ds(c * E_PAD + base, CHUNK)], colv)
        pltpu.sync_copy(rowp_hbm.at[pl.ds(base, CHUNK)], rowv)
        pltpu.async_copy(table_hbm.at[colv], rowsv, sem).wait()
        pltpu.sync_copy(rowsv, acc.at[rowv], add=True)

        @pl.when(lax.rem(ch, 2) == c)
        def _hist():
            for i in range(CHUNK // L):
                c16 = colv[pl.ds(i * L, L)] - coff
                r16 = rowv[pl.ds(i * L, L)]
                d16 = plsc.load_gather(disv, [c16])
                plsc.addupdate_scatter(histv, [r16], d16)

        return carry

    lax.fori_loop(0, NCH_MAIN, cbody, 0)
    plsc.subcore_barrier()
    pltpu.sync_copy(acc.at[pl.ds(s * RPS, RPS)],
                    sums_hbm.at[pl.ds(c * NH + s * RPS, RPS)])
    pltpu.sync_copy(histv, s1p_hbm.at[c * NS + s])


def _final_body(s2_ref, s3_ref, s1p_ref, dis_ref, wt_ref, b_ref, o_ref):
    s1 = jnp.sum(s1p_ref[...], axis=0)
    num = dis_ref[...] * s1[:, None] * s2_ref[...]
    ratio = jnp.nan_to_num(num / s3_ref[...])
    o_ref[...] = (
        jnp.dot(ratio, wt_ref[...], preferred_element_type=jnp.float32)
        + b_ref[...]
    )


_final = pl.pallas_call(
    _final_body,
    grid=(NH // RN,),
    in_specs=[
        pl.BlockSpec((RN, D), lambda i: (i, 0)),
        pl.BlockSpec((RN, D), lambda i: (i, 0)),
        pl.BlockSpec((NC * NS, RN), lambda i: (0, i)),
        pl.BlockSpec((RN, D), lambda i: (i, 0)),
        pl.BlockSpec((D, D), lambda i: (0, 0)),
        pl.BlockSpec((1, D), lambda i: (0, 0)),
    ],
    out_specs=pl.BlockSpec((RN, D), lambda i: (i, 0)),
    out_shape=jax.ShapeDtypeStruct((NH, D), jnp.float32),
)


def kernel(x, edge_index, mask, W, b):
    row = edge_index[0]
    col = edge_index[1]
    pad = E_PAD - E
    colp = jnp.concatenate([col, jnp.full((pad,), N, jnp.int32)])
    rowp = jnp.concatenate([row, jnp.zeros((pad,), jnp.int32)])
    col2 = jnp.concatenate([colp, colp + NH])

    xp = jnp.pad(x, ((0, NH - N), (0, 0)))
    maskp = jnp.pad(mask, ((0, NH - N), (0, 0)))

    degp = _deg_kernel(colp)
    a, bm, discol = _prep(degp, xp, maskp)

    table = jnp.concatenate([a, bm], axis=0)
    dis1dp = jnp.where(jnp.arange(NH) < N, discol[:, 0], 0.0)
    zeros_init = jnp.zeros((NH, D), jnp.float32)

    sums, s1p = _main_kernel(col2, rowp, table, dis1dp, zeros_init)
    out = _final(sums[:NH], sums[NH:], s1p, discol, W.T, b[None])
    return out[:N]


# speedup vs baseline: 11.9561x; 1.0536x over previous
import functools

import jax
import jax.numpy as jnp
from jax import lax
from jax.experimental import pallas as pl
from jax.experimental.pallas import tpu as pltpu
from jax.experimental.pallas import tpu_sc as plsc

N = 10000
E = 320000
D = 128

NC = 2
NS = 16
L = 16

CHUNK = 128
E_PAD = 327680
NROWS = E_PAD // CHUNK
NCH_DEG = E_PAD // (NC * NS * CHUNK)
NCH_MAIN = E_PAD // (NS * CHUNK)
QCH = NCH_MAIN // 4
NH = 10240
RPS = NH // NS
RN = 2048

_mesh = plsc.VectorSubcoreMesh(core_axis_name="c", subcore_axis_name="s")


@functools.partial(
    pl.kernel,
    out_type=jax.ShapeDtypeStruct((NC * NS, NH), jnp.float32),
    mesh=_mesh,
    scratch_types=[
        pltpu.VMEM((NCH_DEG, 1, CHUNK), jnp.int32),
        pltpu.VMEM((NH,), jnp.float32),
    ],
    compiler_params=pltpu.CompilerParams(needs_layout_passes=False),
)
def _deg_kernel(col2_hbm, degp_hbm, colvall, hist):
    c = lax.axis_index("c")
    s = lax.axis_index("s")
    wid = c * NS + s

    pltpu.sync_copy(col2_hbm.at[pl.ds(wid * NCH_DEG, NCH_DEG)], colvall)

    def zbody(i, carry):
        hist[pl.ds(i * L, L)] = jnp.zeros((L,), jnp.float32)
        return carry

    lax.fori_loop(0, NH // L, zbody, 0)

    ones16 = jnp.ones((L,), jnp.float32)

    def cbody(ch, carry):
        for i in range(CHUNK // L):
            c16 = colvall[ch, 0, pl.ds(i * L, L)]
            plsc.addupdate_scatter(hist, [c16], ones16)
        return carry

    lax.fori_loop(0, NCH_DEG, cbody, 0)
    pltpu.sync_copy(hist, degp_hbm.at[wid])


def _prep_body(degp_ref, x_ref, mask_ref, a_ref, b_ref, dis_ref):
    deg = jnp.sum(degp_ref[...], axis=0)
    dis = jnp.where(deg > 0, lax.rsqrt(deg), 0.0)
    disc = dis[:, None]
    xc = x_ref[...]
    xc = jnp.where(jnp.isnan(xc), 0.0, xc)
    m = mask_ref[...]
    a_ref[...] = disc * m * xc
    b_ref[...] = disc * m
    dis_ref[...] = jnp.broadcast_to(disc, (RN, D))


_prep = pl.pallas_call(
    _prep_body,
    grid=(NH // RN,),
    in_specs=[
        pl.BlockSpec((NC * NS, RN), lambda i: (0, i)),
        pl.BlockSpec((RN, D), lambda i: (i, 0)),
        pl.BlockSpec((RN, D), lambda i: (i, 0)),
    ],
    out_specs=[
        pl.BlockSpec((RN, D), lambda i: (i, 0)),
        pl.BlockSpec((RN, D), lambda i: (i, 0)),
        pl.BlockSpec((RN, D), lambda i: (i, 0)),
    ],
    out_shape=[
        jax.ShapeDtypeStruct((NH, D), jnp.float32),
        jax.ShapeDtypeStruct((NH, D), jnp.float32),
        jax.ShapeDtypeStruct((NH, D), jnp.float32),
    ],
)


@functools.partial(
    pl.kernel,
    out_type=jax.ShapeDtypeStruct((NC * NS, NH), jnp.float32),
    mesh=_mesh,
    scratch_types=[
        pltpu.VMEM((NCH_DEG, 1, CHUNK), jnp.int32),
        pltpu.VMEM((NCH_DEG, 1, CHUNK), jnp.int32),
        pltpu.VMEM((NH,), jnp.float32),
        pltpu.VMEM((NH,), jnp.float32),
    ],
    compiler_params=pltpu.CompilerParams(needs_layout_passes=False),
)
def _s1_kernel(col2_hbm, rowp2_hbm, dis_hbm, s1p_hbm,
               colvall, rowvall, disv, histv):
    c = lax.axis_index("c")
    s = lax.axis_index("s")
    wid = c * NS + s

    pltpu.sync_copy(col2_hbm.at[pl.ds(wid * NCH_DEG, NCH_DEG)], colvall)
    pltpu.sync_copy(rowp2_hbm.at[pl.ds(wid * NCH_DEG, NCH_DEG)], rowvall)
    pltpu.sync_copy(dis_hbm, disv)

    def zbody(i, carry):
        histv[pl.ds(i * L, L)] = jnp.zeros((L,), jnp.float32)
        return carry

    lax.fori_loop(0, NH // L, zbody, 0)

    def cbody(ch, carry):
        for i in range(CHUNK // L):
            c16 = colvall[ch, 0, pl.ds(i * L, L)]
            r16 = rowvall[ch, 0, pl.ds(i * L, L)]
            d16 = plsc.load_gather(disv, [c16])
            plsc.addupdate_scatter(histv, [r16], d16)
        return carry

    lax.fori_loop(0, NCH_DEG, cbody, 0)
    pltpu.sync_copy(histv, s1p_hbm.at[wid])


@functools.partial(
    pl.kernel,
    out_type=jax.ShapeDtypeStruct((2 * NH, D), jnp.float32),
    mesh=_mesh,
    scratch_types=[
        pltpu.VMEM_SHARED((NH, D), jnp.float32),
        pltpu.VMEM((QCH, 1, CHUNK), jnp.int32),
        pltpu.VMEM((QCH, 1, CHUNK), jnp.int32),
        pltpu.VMEM((2, CHUNK, D), jnp.float32),
        pltpu.SemaphoreType.DMA,
        pltpu.SemaphoreType.DMA,
    ],
    compiler_params=pltpu.CompilerParams(needs_layout_passes=False),
)
def _main_kernel(col2_hbm, rowp2_hbm, table_hbm, zeros_hbm,
                 sums_hbm,
                 acc, colvall, rowvall, rowsv, sem0, sem1):
    c = lax.axis_index("c")
    s = lax.axis_index("s")
    sems = (sem0, sem1)

    pltpu.sync_copy(zeros_hbm.at[pl.ds(s * RPS, RPS)],
                    acc.at[pl.ds(s * RPS, RPS)])
    plsc.subcore_barrier()

    def process(ch, j, fire_next):
        pltpu.make_async_copy(
            table_hbm.at[colvall.at[ch, 0]], rowsv.at[j], sems[j]).wait()
        pltpu.sync_copy(rowsv.at[j], acc.at[rowvall.at[ch, 0]], add=True)
        if fire_next:
            pltpu.async_copy(
                table_hbm.at[colvall.at[ch + 2, 0]], rowsv.at[j], sems[j])

    def gbody(g, carry):
        process(2 * g, 0, True)
        process(2 * g + 1, 1, True)
        return carry

    for h in range(NCH_MAIN // QCH):
        pltpu.sync_copy(
            col2_hbm.at[pl.ds(c * NROWS + s * NCH_MAIN + h * QCH, QCH)],
            colvall)
        pltpu.sync_copy(
            rowp2_hbm.at[pl.ds(s * NCH_MAIN + h * QCH, QCH)], rowvall)
        pltpu.async_copy(table_hbm.at[colvall.at[0, 0]], rowsv.at[0], sem0)
        pltpu.async_copy(table_hbm.at[colvall.at[1, 0]], rowsv.at[1], sem1)
        lax.fori_loop(0, QCH // 2 - 1, gbody, 0)
        process(QCH - 2, 0, False)
        process(QCH - 1, 1, False)

    plsc.subcore_barrier()
    pltpu.sync_copy(acc.at[pl.ds(s * RPS, RPS)],
                    sums_hbm.at[pl.ds(c * NH + s * RPS, RPS)])


def _final_body(s2_ref, s3_ref, s1p_ref, dis_ref, wt_ref, b_ref, o_ref):
    s1 = jnp.sum(s1p_ref[...], axis=0)
    num = dis_ref[...] * s1[:, None] * s2_ref[...]
    ratio = jnp.nan_to_num(num / s3_ref[...])
    o_ref[...] = (
        jnp.dot(ratio, wt_ref[...], preferred_element_type=jnp.float32)
        + b_ref[...]
    )


_final = pl.pallas_call(
    _final_body,
    grid=(NH // RN,),
    in_specs=[
        pl.BlockSpec((RN, D), lambda i: (i, 0)),
        pl.BlockSpec((RN, D), lambda i: (i, 0)),
        pl.BlockSpec((NC * NS, RN), lambda i: (0, i)),
        pl.BlockSpec((RN, D), lambda i: (i, 0)),
        pl.BlockSpec((D, D), lambda i: (0, 0)),
        pl.BlockSpec((1, D), lambda i: (0, 0)),
    ],
    out_specs=pl.BlockSpec((RN, D), lambda i: (i, 0)),
    out_shape=jax.ShapeDtypeStruct((NH, D), jnp.float32),
)


def kernel(x, edge_index, mask, W, b):
    row = edge_index[0]
    col = edge_index[1]
    pad = E_PAD - E
    colp = jnp.concatenate([col, jnp.full((pad,), N, jnp.int32)])
    rowp = jnp.concatenate([row, jnp.zeros((pad,), jnp.int32)])
    col2 = jnp.concatenate([colp, colp + NH]).reshape(2 * NROWS, 1, CHUNK)
    rowp2 = rowp.reshape(NROWS, 1, CHUNK)

    xp = jnp.pad(x, ((0, NH - N), (0, 0)))
    maskp = jnp.pad(mask, ((0, NH - N), (0, 0)))

    degp = _deg_kernel(col2)
    a, bm, discol = _prep(degp, xp, maskp)

    table = jnp.concatenate([a, bm], axis=0)
    dis1dp = jnp.where(jnp.arange(NH) < N, discol[:, 0], 0.0)
    zeros_init = jnp.zeros((NH, D), jnp.float32)

    s1p = _s1_kernel(col2, rowp2, dis1dp)
    sums = _main_kernel(col2, rowp2, table, zeros_init)
    out = _final(sums[:NH], sums[NH:], s1p, discol, W.T, b[None])
    return out[:N]


# speedup vs baseline: 12.6194x; 1.0555x over previous
import functools

import jax
import jax.numpy as jnp
from jax import lax
from jax.experimental import pallas as pl
from jax.experimental.pallas import tpu as pltpu
from jax.experimental.pallas import tpu_sc as plsc

N = 10000
E = 320000
D = 128

NC = 2
NS = 16
L = 16

CHUNK = 128
E_PAD = 327680
NROWS = E_PAD // CHUNK
NCH_DEG = E_PAD // (NC * NS * CHUNK)
NCH_MAIN = E_PAD // (NS * CHUNK)
NCHM = E_PAD // (NC * NS * CHUNK)
HCH = NCHM // 2
DC = 2 * D
NH = 10240
RPS = NH // NS
RN = 2048

_mesh = plsc.VectorSubcoreMesh(core_axis_name="c", subcore_axis_name="s")


@functools.partial(
    pl.kernel,
    out_type=jax.ShapeDtypeStruct((NC * NS, NH), jnp.float32),
    mesh=_mesh,
    scratch_types=[
        pltpu.VMEM((NCH_DEG, 1, CHUNK), jnp.int32),
        pltpu.VMEM((NH,), jnp.float32),
    ],
    compiler_params=pltpu.CompilerParams(needs_layout_passes=False),
)
def _deg_kernel(col2_hbm, degp_hbm, colvall, hist):
    c = lax.axis_index("c")
    s = lax.axis_index("s")
    wid = c * NS + s

    pltpu.sync_copy(col2_hbm.at[pl.ds(wid * NCH_DEG, NCH_DEG)], colvall)

    def zbody(i, carry):
        hist[pl.ds(i * L, L)] = jnp.zeros((L,), jnp.float32)
        return carry

    lax.fori_loop(0, NH // L, zbody, 0)

    ones16 = jnp.ones((L,), jnp.float32)

    def cbody(ch, carry):
        for i in range(CHUNK // L):
            c16 = colvall[ch, 0, pl.ds(i * L, L)]
            plsc.addupdate_scatter(hist, [c16], ones16)
        return carry

    lax.fori_loop(0, NCH_DEG, cbody, 0)
    pltpu.sync_copy(hist, degp_hbm.at[wid])


def _prep_body(degp_ref, x_ref, mask_ref, tab_ref, dis_ref):
    deg = jnp.sum(degp_ref[...], axis=0)
    dis = jnp.where(deg > 0, lax.rsqrt(deg), 0.0)
    disc = dis[:, None]
    xc = x_ref[...]
    xc = jnp.where(jnp.isnan(xc), 0.0, xc)
    m = mask_ref[...]
    a = disc * m * xc
    bm = disc * m
    tab_ref[...] = jnp.concatenate([a, bm], axis=1).astype(jnp.bfloat16)
    dis_ref[...] = jnp.broadcast_to(disc, (RN, D))


_prep = pl.pallas_call(
    _prep_body,
    grid=(NH // RN,),
    in_specs=[
        pl.BlockSpec((NC * NS, RN), lambda i: (0, i)),
        pl.BlockSpec((RN, D), lambda i: (i, 0)),
        pl.BlockSpec((RN, D), lambda i: (i, 0)),
    ],
    out_specs=[
        pl.BlockSpec((RN, DC), lambda i: (i, 0)),
        pl.BlockSpec((RN, D), lambda i: (i, 0)),
    ],
    out_shape=[
        jax.ShapeDtypeStruct((NH, DC), jnp.bfloat16),
        jax.ShapeDtypeStruct((NH, D), jnp.float32),
    ],
)


@functools.partial(
    pl.kernel,
    out_type=jax.ShapeDtypeStruct((NC * NS, NH), jnp.float32),
    mesh=_mesh,
    scratch_types=[
        pltpu.VMEM((NCH_DEG, 1, CHUNK), jnp.int32),
        pltpu.VMEM((NCH_DEG, 1, CHUNK), jnp.int32),
        pltpu.VMEM((NH,), jnp.float32),
        pltpu.VMEM((NH,), jnp.float32),
    ],
    compiler_params=pltpu.CompilerParams(needs_layout_passes=False),
)
def _s1_kernel(col2_hbm, rowp2_hbm, dis_hbm, s1p_hbm,
               colvall, rowvall, disv, histv):
    c = lax.axis_index("c")
    s = lax.axis_index("s")
    wid = c * NS + s

    pltpu.sync_copy(col2_hbm.at[pl.ds(wid * NCH_DEG, NCH_DEG)], colvall)
    pltpu.sync_copy(rowp2_hbm.at[pl.ds(wid * NCH_DEG, NCH_DEG)], rowvall)
    pltpu.sync_copy(dis_hbm, disv)

    def zbody(i, carry):
        histv[pl.ds(i * L, L)] = jnp.zeros((L,), jnp.float32)
        return carry

    lax.fori_loop(0, NH // L, zbody, 0)

    def cbody(ch, carry):
        for i in range(CHUNK // L):
            c16 = colvall[ch, 0, pl.ds(i * L, L)]
            r16 = rowvall[ch, 0, pl.ds(i * L, L)]
            d16 = plsc.load_gather(disv, [c16])
            plsc.addupdate_scatter(histv, [r16], d16)
        return carry

    lax.fori_loop(0, NCH_DEG, cbody, 0)
    pltpu.sync_copy(histv, s1p_hbm.at[wid])


@functools.partial(
    pl.kernel,
    out_type=jax.ShapeDtypeStruct((2 * NH, 2, D), jnp.bfloat16),
    mesh=_mesh,
    scratch_types=[
        pltpu.VMEM_SHARED((NH, 2, D), jnp.bfloat16),
        pltpu.VMEM((HCH, 1, CHUNK), jnp.int32),
        pltpu.VMEM((HCH, 1, CHUNK), jnp.int32),
        pltpu.VMEM((2, CHUNK, 2, D), jnp.bfloat16),
        pltpu.SemaphoreType.DMA,
        pltpu.SemaphoreType.DMA,
    ],
    compiler_params=pltpu.CompilerParams(
        needs_layout_passes=False, use_tc_tiling_on_sc=False),
)
def _main_kernel(col2_hbm, rowp2_hbm, table_hbm, zeros_hbm,
                 sums_hbm,
                 acc, colvall, rowvall, rowsv, sem0, sem1):
    c = lax.axis_index("c")
    s = lax.axis_index("s")
    sems = (sem0, sem1)

    pltpu.sync_copy(zeros_hbm.at[pl.ds(s * RPS, RPS)],
                    acc.at[pl.ds(s * RPS, RPS)])
    plsc.subcore_barrier()

    def process(ch, j, fire_next):
        pltpu.make_async_copy(
            table_hbm.at[colvall.at[ch, 0]], rowsv.at[j], sems[j]).wait()
        pltpu.sync_copy(rowsv.at[j], acc.at[rowvall.at[ch, 0]], add=True)
        if fire_next:
            pltpu.async_copy(
                table_hbm.at[colvall.at[ch + 2, 0]], rowsv.at[j], sems[j])

    def gbody(g, carry):
        process(2 * g, 0, True)
        process(2 * g + 1, 1, True)
        return carry

    wid = c * NS + s
    for h in range(NCHM // HCH):
        pltpu.sync_copy(
            col2_hbm.at[pl.ds(wid * NCHM + h * HCH, HCH)], colvall)
        pltpu.sync_copy(
            rowp2_hbm.at[pl.ds(wid * NCHM + h * HCH, HCH)], rowvall)
        pltpu.async_copy(table_hbm.at[colvall.at[0, 0]], rowsv.at[0], sem0)
        pltpu.async_copy(table_hbm.at[colvall.at[1, 0]], rowsv.at[1], sem1)
        lax.fori_loop(0, HCH // 2 - 1, gbody, 0)
        process(HCH - 2, 0, False)
        process(HCH - 1, 1, False)

    plsc.subcore_barrier()
    pltpu.sync_copy(acc.at[pl.ds(s * RPS, RPS)],
                    sums_hbm.at[pl.ds(c * NH + s * RPS, RPS)])


def _final_body(p0_ref, p1_ref, s1p_ref, dis_ref, wt_ref, b_ref, o_ref):
    p = (p0_ref[...].astype(jnp.float32) + p1_ref[...].astype(jnp.float32))
    s2 = p[:, :D]
    s3 = p[:, D:]
    s1 = jnp.sum(s1p_ref[...], axis=0)
    num = dis_ref[...] * s1[:, None] * s2
    ratio = jnp.nan_to_num(num / s3)
    o_ref[...] = (
        jnp.dot(ratio, wt_ref[...], preferred_element_type=jnp.float32)
        + b_ref[...]
    )


_final = pl.pallas_call(
    _final_body,
    grid=(NH // RN,),
    in_specs=[
        pl.BlockSpec((RN, DC), lambda i: (i, 0)),
        pl.BlockSpec((RN, DC), lambda i: (i, 0)),
        pl.BlockSpec((NC * NS, RN), lambda i: (0, i)),
        pl.BlockSpec((RN, D), lambda i: (i, 0)),
        pl.BlockSpec((D, D), lambda i: (0, 0)),
        pl.BlockSpec((1, D), lambda i: (0, 0)),
    ],
    out_specs=pl.BlockSpec((RN, D), lambda i: (i, 0)),
    out_shape=jax.ShapeDtypeStruct((NH, D), jnp.float32),
)


def kernel(x, edge_index, mask, W, b):
    row = edge_index[0]
    col = edge_index[1]
    pad = E_PAD - E
    colp = jnp.concatenate([col, jnp.full((pad,), N, jnp.int32)])
    rowp = jnp.concatenate([row, jnp.zeros((pad,), jnp.int32)])
    col2 = colp.reshape(NROWS, 1, CHUNK)
    rowp2 = rowp.reshape(NROWS, 1, CHUNK)

    xp = jnp.pad(x, ((0, NH - N), (0, 0)))
    maskp = jnp.pad(mask, ((0, NH - N), (0, 0)))

    degp = _deg_kernel(col2)
    table, discol = _prep(degp, xp, maskp)

    dis1dp = jnp.where(jnp.arange(NH) < N, discol[:, 0], 0.0)
    zeros_init = jnp.zeros((NH, 2, D), jnp.bfloat16)

    s1p = _s1_kernel(col2, rowp2, dis1dp)
    sums = _main_kernel(col2, rowp2, table.reshape(NH, 2, D),
                        zeros_init).reshape(2 * NH, DC)
    out = _final(sums[:NH], sums[NH:], s1p, discol, W.T, b[None])
    return out[:N]
